# baseline (device time: 31065 ns/iter reference)
import jax
import jax.numpy as jnp
from jax import lax
from jax.experimental import pallas as pl
from jax.experimental.pallas import tpu as pltpu

NZ = 4
ROWS = 1024
COLS = 512
P = 288
DR, DC = 8, 128


def kernel(x, dest):
    def body(x_ref, d_ref, out_ref,
             sb, cnt_s, rb, cgv, cgs,
             send_b, recv_b, send_c, recv_c, cp_sem):
        my_x = lax.axis_index("x")
        my_y = lax.axis_index("y")
        me = lax.axis_index("z")

        barrier = pltpu.get_barrier_semaphore()
        for k in range(1, NZ):
            pl.semaphore_signal(
                barrier, inc=1,
                device_id=(my_x, my_y, (me + k) % NZ),
                device_id_type=pl.DeviceIdType.MESH,
            )
        pl.semaphore_wait(barrier, NZ - 1)

        lane = lax.broadcasted_iota(jnp.int32, (ROWS, DC), 1)
        oh = (d_ref[...] == lane).astype(jnp.int32)

        def cumsum0(a):
            out = a
            sh = 1
            while sh < ROWS:
                shifted = jnp.concatenate(
                    [jnp.zeros((sh, DC), a.dtype),
                     lax.slice(out, (0, 0), (ROWS - sh, DC))], axis=0)
                out = out + shifted
                sh *= 2
            return out

        csum = cumsum0(oh)
        pos = (csum * oh).sum(axis=1, keepdims=True) - 1
        cnt_s[...] = lax.slice(csum, (ROWS - DR, 0), (ROWS, DC))

        cnt_rdmas = []
        for k in range(1, NZ):
            tgt = (my_x, my_y, (me + k) % NZ)
            rc = pltpu.make_async_remote_copy(
                src_ref=cnt_s,
                dst_ref=cgv.at[me],
                send_sem=send_c.at[k - 1],
                recv_sem=recv_c.at[k - 1],
                device_id=tgt, device_id_type=pl.DeviceIdType.MESH,
            )
            rc.start()
            cnt_rdmas.append(rc)

        iota_p = lax.broadcasted_iota(jnp.int32, (ROWS, P), 1)

        def block_for(tz):
            sel = (pos == iota_p) & (d_ref[...] == tz)
            return lax.dot_general(
                sel.astype(jnp.float32), x_ref[...],
                (((0,), (0,)), ((), ())),
                preferred_element_type=jnp.float32,
            )

        blk_rdmas = []
        for k in range(1, NZ):
            tz = (me + k) % NZ
            tgt = (my_x, my_y, tz)
            sb[pl.ds(k - 1, 1), :, :] = block_for(tz)[None]
            rblk = pltpu.make_async_remote_copy(
                src_ref=sb.at[k - 1],
                dst_ref=rb.at[me],
                send_sem=send_b.at[k - 1],
                recv_sem=recv_b.at[me],
                device_id=tgt, device_id_type=pl.DeviceIdType.MESH,
            )
            rblk.start()
            blk_rdmas.append(rblk)

        own_block = block_for(me)

        for rc in cnt_rdmas:
            rc.wait_recv()
        cgv[pl.ds(me, 1), :, :] = cnt_s[...][None]
        cp = pltpu.make_async_copy(cgv, cgs, cp_sem)
        cp.start()
        cp.wait()
        c_sc = [cgs[s, DR - 1, me] for s in range(NZ)]
        o_sc = [jnp.int32(0)]
        for s in range(1, NZ):
            o_sc.append(o_sc[-1] + c_sc[s - 1])

        def pick(vals, s):
            acc = vals[0] * 0
            for i in range(NZ):
                acc = acc + jnp.where(s == i, vals[i], 0)
            return acc

        t_col = lax.broadcasted_iota(jnp.int32, (ROWS, 1), 0)
        p_row = lax.broadcasted_iota(jnp.int32, (1, P), 1)

        def assemble(s, block):
            c_s = pick(c_sc, s)
            o_s = pick(o_sc, s)
            ti = jnp.where(p_row < c_s, o_s + p_row, jnp.int32(2 * ROWS))
            q = (t_col == ti).astype(jnp.float32)
            return lax.dot_general(
                q, block, (((1,), (0,)), ((), ())),
                preferred_element_type=jnp.float32,
            )

        acc = assemble(me, own_block)

        s1 = jnp.abs(me - 1)
        s2 = jnp.where(me == 0, 2, jnp.where(me == 3, 1, me + 1))
        s3 = jnp.where(me <= 1, 3, 0)
        for s_k in (s1, s2, s3):
            rwait = pltpu.make_async_remote_copy(
                src_ref=rb.at[s_k], dst_ref=rb.at[s_k],
                send_sem=send_b.at[0], recv_sem=recv_b.at[s_k],
                device_id=(my_x, my_y, s_k),
                device_id_type=pl.DeviceIdType.MESH,
            )
            rwait.wait_recv()
            val = rb[pl.ds(s_k, 1), :, :].reshape(P, COLS)
            acc = acc + assemble(s_k, val)
        out_ref[...] = acc

        for rblk in blk_rdmas:
            rblk.wait_send()
        for rc in cnt_rdmas:
            rc.wait_send()

    return pl.pallas_call(
        body,
        out_shape=jax.ShapeDtypeStruct((ROWS, COLS), jnp.float32),
        in_specs=[
            pl.BlockSpec(memory_space=pltpu.VMEM),
            pl.BlockSpec(memory_space=pltpu.VMEM),
        ],
        out_specs=pl.BlockSpec(memory_space=pltpu.VMEM),
        scratch_shapes=[
            pltpu.VMEM((NZ - 1, P, COLS), jnp.float32),
            pltpu.VMEM((DR, DC), jnp.int32),
            pltpu.VMEM((NZ, P, COLS), jnp.float32),
            pltpu.VMEM((NZ, DR, DC), jnp.int32),
            pltpu.SMEM((NZ, DR, DC), jnp.int32),
            pltpu.SemaphoreType.DMA((NZ - 1,)),
            pltpu.SemaphoreType.DMA((NZ,)),
            pltpu.SemaphoreType.DMA((NZ - 1,)),
            pltpu.SemaphoreType.DMA((NZ - 1,)),
            pltpu.SemaphoreType.DMA,
        ],
        compiler_params=pltpu.CompilerParams(collective_id=0),
    )(x, dest.astype(jnp.int32)[:, None])


# device time: 31039 ns/iter; 1.0008x vs baseline; 1.0008x over previous
import jax
import jax.numpy as jnp
from jax import lax
from jax.experimental import pallas as pl
from jax.experimental.pallas import tpu as pltpu

NZ = 4
ROWS = 1024
COLS = 512
P = 288
DR, DC = 8, 128


def kernel(x, dest):
    def body(x_ref, d_ref, out_ref,
             sb, cnt_s, rb, cgv, cgs,
             send_b, recv_b, send_c, recv_c, cp_sem):
        my_x = lax.axis_index("x")
        my_y = lax.axis_index("y")
        me = lax.axis_index("z")

        barrier = pltpu.get_barrier_semaphore()
        for k in range(1, NZ):
            pl.semaphore_signal(
                barrier, inc=1,
                device_id=(my_x, my_y, (me + k) % NZ),
                device_id_type=pl.DeviceIdType.MESH,
            )
        pl.semaphore_wait(barrier, NZ - 1)

        lane = lax.broadcasted_iota(jnp.int32, (ROWS, DC), 1)
        oh = (d_ref[...] == lane).astype(jnp.int32)

        def cumsum0(a):
            out = a
            sh = 1
            while sh < ROWS:
                shifted = jnp.concatenate(
                    [jnp.zeros((sh, DC), a.dtype),
                     lax.slice(out, (0, 0), (ROWS - sh, DC))], axis=0)
                out = out + shifted
                sh *= 2
            return out

        csum = cumsum0(oh)
        pos = (csum * oh).sum(axis=1, keepdims=True) - 1
        cnt_s[...] = lax.slice(csum, (ROWS - DR, 0), (ROWS, DC))

        cnt_rdmas = []
        for k in range(1, NZ):
            tgt = (my_x, my_y, (me + k) % NZ)
            rc = pltpu.make_async_remote_copy(
                src_ref=cnt_s,
                dst_ref=cgv.at[me],
                send_sem=send_c.at[k - 1],
                recv_sem=recv_c.at[k - 1],
                device_id=tgt, device_id_type=pl.DeviceIdType.MESH,
            )
            rc.start()
            cnt_rdmas.append(rc)

        iota_p = lax.broadcasted_iota(jnp.int32, (ROWS, P), 1)

        def block_for(tz):
            sel = (pos == iota_p) & (d_ref[...] == tz)
            return lax.dot_general(
                sel.astype(jnp.float32), x_ref[...],
                (((0,), (0,)), ((), ())),
                preferred_element_type=jnp.float32,
            )

        blk_rdmas = []
        for k in range(1, NZ):
            tz = (me + k) % NZ
            tgt = (my_x, my_y, tz)
            sb[pl.ds(k - 1, 1), :, :] = block_for(tz)[None]
            rblk = pltpu.make_async_remote_copy(
                src_ref=sb.at[k - 1],
                dst_ref=rb.at[me],
                send_sem=send_b.at[k - 1],
                recv_sem=recv_b.at[k - 1],
                device_id=tgt, device_id_type=pl.DeviceIdType.MESH,
            )
            rblk.start()
            blk_rdmas.append(rblk)

        own_block = block_for(me)

        for rc in cnt_rdmas:
            rc.wait_recv()
        cgv[pl.ds(me, 1), :, :] = cnt_s[...][None]
        cp = pltpu.make_async_copy(cgv, cgs, cp_sem)
        cp.start()
        cp.wait()
        c_sc = [cgs[s, DR - 1, me] for s in range(NZ)]
        o_sc = [jnp.int32(0)]
        for s in range(1, NZ):
            o_sc.append(o_sc[-1] + c_sc[s - 1])

        def pick(vals, s):
            acc = vals[0] * 0
            for i in range(NZ):
                acc = acc + jnp.where(s == i, vals[i], 0)
            return acc

        t_col = lax.broadcasted_iota(jnp.int32, (ROWS, 1), 0)
        p_row = lax.broadcasted_iota(jnp.int32, (1, P), 1)

        def assemble(s, block):
            c_s = pick(c_sc, s)
            o_s = pick(o_sc, s)
            ti = jnp.where(p_row < c_s, o_s + p_row, jnp.int32(2 * ROWS))
            q = (t_col == ti).astype(jnp.float32)
            return lax.dot_general(
                q, block, (((1,), (0,)), ((), ())),
                preferred_element_type=jnp.float32,
            )

        acc = assemble(me, own_block)
        for k in range(1, NZ):
            blk_rdmas[k - 1].wait_recv()
            s_k = (me - k) % NZ
            val = rb[pl.ds(s_k, 1), :, :].reshape(P, COLS)
            acc = acc + assemble(s_k, val)
        out_ref[...] = acc

        for rblk in blk_rdmas:
            rblk.wait_send()
        for rc in cnt_rdmas:
            rc.wait_send()

    return pl.pallas_call(
        body,
        out_shape=jax.ShapeDtypeStruct((ROWS, COLS), jnp.float32),
        in_specs=[
            pl.BlockSpec(memory_space=pltpu.VMEM),
            pl.BlockSpec(memory_space=pltpu.VMEM),
        ],
        out_specs=pl.BlockSpec(memory_space=pltpu.VMEM),
        scratch_shapes=[
            pltpu.VMEM((NZ - 1, P, COLS), jnp.float32),
            pltpu.VMEM((DR, DC), jnp.int32),
            pltpu.VMEM((NZ, P, COLS), jnp.float32),
            pltpu.VMEM((NZ, DR, DC), jnp.int32),
            pltpu.SMEM((NZ, DR, DC), jnp.int32),
            pltpu.SemaphoreType.DMA((NZ - 1,)),
            pltpu.SemaphoreType.DMA((NZ - 1,)),
            pltpu.SemaphoreType.DMA((NZ - 1,)),
            pltpu.SemaphoreType.DMA((NZ - 1,)),
            pltpu.SemaphoreType.DMA,
        ],
        compiler_params=pltpu.CompilerParams(collective_id=0),
    )(x, dest.astype(jnp.int32)[:, None])


# device time: 31018 ns/iter; 1.0015x vs baseline; 1.0007x over previous
import jax
import jax.numpy as jnp
from jax import lax
from jax.experimental import pallas as pl
from jax.experimental.pallas import tpu as pltpu

NZ = 4
ROWS = 1024
COLS = 512
P = 288
DR, DC = 8, 128


def kernel(x, dest):
    def body(x_ref, d_ref, out_ref,
             sb, cnt_s, rb, cgv, cgs,
             send_b, recv_b, send_c, recv_c, cp_sem):
        my_x = lax.axis_index("x")
        my_y = lax.axis_index("y")
        me = lax.axis_index("z")

        barrier = pltpu.get_barrier_semaphore()
        for k in range(1, NZ):
            pl.semaphore_signal(
                barrier, inc=1,
                device_id=(my_x, my_y, (me + k) % NZ),
                device_id_type=pl.DeviceIdType.MESH,
            )
        pl.semaphore_wait(barrier, NZ - 1)

        lane = lax.broadcasted_iota(jnp.int32, (ROWS, DC), 1)
        oh = (d_ref[...] == lane).astype(jnp.int32)

        def cumsum0(a):
            out = a
            sh = 1
            while sh < ROWS:
                shifted = jnp.concatenate(
                    [jnp.zeros((sh, DC), a.dtype),
                     lax.slice(out, (0, 0), (ROWS - sh, DC))], axis=0)
                out = out + shifted
                sh *= 2
            return out

        csum = cumsum0(oh)
        pos = (csum * oh).sum(axis=1, keepdims=True) - 1
        cnt_s[...] = lax.slice(csum, (ROWS - DR, 0), (ROWS, DC))

        cnt_rdmas = []
        for k in range(1, NZ):
            tgt = (my_x, my_y, (me + k) % NZ)
            rc = pltpu.make_async_remote_copy(
                src_ref=cnt_s,
                dst_ref=cgv.at[me],
                send_sem=send_c.at[k - 1],
                recv_sem=recv_c.at[k - 1],
                device_id=tgt, device_id_type=pl.DeviceIdType.MESH,
            )
            rc.start()
            cnt_rdmas.append(rc)

        iota_p = lax.broadcasted_iota(jnp.int32, (ROWS, P), 1)

        def block_for(tz):
            sel = (pos == iota_p) & (d_ref[...] == tz)
            return lax.dot_general(
                sel.astype(jnp.float32), x_ref[...],
                (((0,), (0,)), ((), ())),
                preferred_element_type=jnp.float32,
            )

        blk_rdmas = []
        for k in range(1, NZ):
            tz = (me + k) % NZ
            tgt = (my_x, my_y, tz)
            sb[pl.ds(k - 1, 1), :, :] = block_for(tz)[None]
            rblk = pltpu.make_async_remote_copy(
                src_ref=sb.at[k - 1],
                dst_ref=rb.at[me],
                send_sem=send_b.at[k - 1],
                recv_sem=recv_b.at[k - 1],
                device_id=tgt, device_id_type=pl.DeviceIdType.MESH,
            )
            rblk.start()
            blk_rdmas.append(rblk)

        own_block = block_for(me)

        for rc in cnt_rdmas:
            rc.wait_recv()
        cgv[pl.ds(me, 1), :, :] = cnt_s[...][None]
        cp = pltpu.make_async_copy(cgv, cgs, cp_sem)
        cp.start()
        cp.wait()
        c_sc = [cgs[s, DR - 1, me] for s in range(NZ)]
        o_sc = [jnp.int32(0)]
        for s in range(1, NZ):
            o_sc.append(o_sc[-1] + c_sc[s - 1])

        def pick(vals, s):
            acc = vals[0] * 0
            for i in range(NZ):
                acc = acc + jnp.where(s == i, vals[i], 0)
            return acc

        t_col = lax.broadcasted_iota(jnp.int32, (ROWS, 1), 0)
        p_row = lax.broadcasted_iota(jnp.int32, (1, P), 1)

        def assemble(s, block):
            c_s = pick(c_sc, s)
            o_s = pick(o_sc, s)
            ti = jnp.where(p_row < c_s, o_s + p_row, jnp.int32(2 * ROWS))
            q = (t_col == ti).astype(jnp.float32)
            return lax.dot_general(
                q, block, (((1,), (0,)), ((), ())),
                preferred_element_type=jnp.float32,
            )

        acc = assemble(me, own_block)
        for k in range(1, NZ):
            blk_rdmas[k - 1].wait_recv()
            s_k = (me - k) % NZ
            val = rb[pl.ds(s_k, 1), :, :].reshape(P, COLS)
            acc = acc + assemble(s_k, val)
        out_ref[...] = acc

        for rblk in blk_rdmas:
            rblk.wait_send()
        for rc in cnt_rdmas:
            rc.wait_send()

    return pl.pallas_call(
        body,
        out_shape=jax.ShapeDtypeStruct((ROWS, COLS), jnp.float32),
        in_specs=[
            pl.BlockSpec(memory_space=pltpu.VMEM),
            pl.BlockSpec(memory_space=pltpu.VMEM),
        ],
        out_specs=pl.BlockSpec(memory_space=pltpu.VMEM),
        scratch_shapes=[
            pltpu.VMEM((NZ - 1, P, COLS), jnp.float32),
            pltpu.VMEM((DR, DC), jnp.int32),
            pltpu.VMEM((NZ, P, COLS), jnp.float32),
            pltpu.VMEM((NZ, DR, DC), jnp.int32),
            pltpu.SMEM((NZ, DR, DC), jnp.int32),
            pltpu.SemaphoreType.DMA((NZ - 1,)),
            pltpu.SemaphoreType.DMA((NZ - 1,)),
            pltpu.SemaphoreType.DMA((NZ - 1,)),
            pltpu.SemaphoreType.DMA((NZ - 1,)),
            pltpu.SemaphoreType.DMA,
        ],
        compiler_params=pltpu.CompilerParams(collective_id=5),
    )(x, dest.astype(jnp.int32)[:, None])
